# baseline (speedup 1.0000x reference)
import functools

import jax
import jax.numpy as jnp
from jax import lax
from jax.experimental import pallas as pl
from jax.experimental.pallas import tpu as pltpu
from jax.experimental.pallas import tpu_sc as plsc

EMBED_DIM = 128
SPATIAL = 32 * 32
BB = 8
BWARM = 128


def _make_sc_gather(B):
    info = plsc.get_sparse_core_info()
    nw = info.num_cores * info.num_subcores
    b_per_w = B // nw
    mesh = plsc.VectorSubcoreMesh(core_axis_name="c", subcore_axis_name="s")

    @functools.partial(
        pl.kernel, mesh=mesh,
        out_type=jax.ShapeDtypeStruct((B, EMBED_DIM), jnp.float32),
        scratch_types=[
            pltpu.VMEM((b_per_w,), jnp.int32),
            pltpu.VMEM((b_per_w, EMBED_DIM), jnp.float32),
            pltpu.SemaphoreType.DMA,
        ],
    )
    def sc_gather(t_hbm, emb_hbm, out_hbm, idx_v, rows_v, sem):
        wid = lax.axis_index("s") * info.num_cores + lax.axis_index("c")
        base = wid * b_per_w
        pltpu.sync_copy(t_hbm.at[pl.ds(base, b_per_w)], idx_v)
        pltpu.async_copy(emb_hbm.at[idx_v], rows_v, sem).wait()
        pltpu.sync_copy(rows_v, out_hbm.at[pl.ds(base, b_per_w)])

    return sc_gather


def _warm_body(t_ref, emb_ref, o_ref, gscr):
    i = pl.program_id(0)
    vpad = emb_ref.shape[0]
    tcol = jnp.stack([t_ref[i * BB + j] for j in range(BB)]).reshape(BB, 1)
    cols = lax.broadcasted_iota(jnp.int32, (BB, vpad), 1)
    onehot = (cols == tcol).astype(jnp.float32)
    gscr[...] = lax.dot_general(
        onehot, emb_ref[...], (((1,), (0,)), ((), ())),
        preferred_element_type=jnp.float32)
    o_ref[...] = jnp.broadcast_to(
        gscr[...][:, None, :], (BB, SPATIAL, EMBED_DIM))


def _main_body(g_ref, alias_ref, o_ref):
    del alias_ref
    i = pl.program_id(0)
    gs = g_ref[pl.ds(BWARM + i * BB, BB), :]
    o_ref[...] = jnp.broadcast_to(gs[:, None, :], (BB, SPATIAL, EMBED_DIM))


def kernel(x, t, embedding):
    B = t.shape[0]
    V = embedding.shape[0]
    H, W = x.shape[-2], x.shape[-1]
    vpad = (V + 7) // 8 * 8
    emb_pad = jnp.pad(embedding, ((0, vpad - V), (0, 0)))

    g = _make_sc_gather(B)(t, embedding)

    warm_spec = pltpu.PrefetchScalarGridSpec(
        num_scalar_prefetch=1,
        grid=(BWARM // BB,),
        in_specs=[pl.BlockSpec((vpad, EMBED_DIM), lambda i, t_r: (0, 0))],
        out_specs=pl.BlockSpec(
            (BB, SPATIAL, EMBED_DIM), lambda i, t_r: (i, 0, 0)),
        scratch_shapes=[pltpu.VMEM((BB, EMBED_DIM), jnp.float32)],
    )
    out1 = pl.pallas_call(
        _warm_body,
        grid_spec=warm_spec,
        out_shape=jax.ShapeDtypeStruct((B, SPATIAL, EMBED_DIM), jnp.float32),
    )(t, emb_pad)

    out = pl.pallas_call(
        _main_body,
        grid=((B - BWARM) // BB,),
        in_specs=[
            pl.BlockSpec((B, EMBED_DIM), lambda i: (0, 0)),
            pl.BlockSpec(memory_space=pl.ANY),
        ],
        out_specs=pl.BlockSpec(
            (BB, SPATIAL, EMBED_DIM), lambda i: (i + BWARM // BB, 0, 0)),
        out_shape=jax.ShapeDtypeStruct((B, SPATIAL, EMBED_DIM), jnp.float32),
        input_output_aliases={1: 0},
    )(g, out1)
    return out.reshape(B, H, W, EMBED_DIM).transpose(0, 3, 1, 2)
